# baseline (speedup 1.0000x reference)
import jax
import jax.numpy as jnp
from jax.experimental import pallas as pl
from jax.experimental.pallas import tpu as pltpu

N = 1024
D = 128
P = 64
NC = 10
K = 10
OUT = 128

_HIGH = jax.lax.Precision.HIGHEST


def _dot(a, b, dims):
    return jax.lax.dot_general(a, b, (dims, ((), ())),
                               preferred_element_type=jnp.float32,
                               precision=_HIGH)


def _bdot(a, b, dims):
    return jax.lax.dot_general(a.astype(jnp.bfloat16), b.astype(jnp.bfloat16),
                               (dims, ((), ())),
                               preferred_element_type=jnp.float32)


SLICES = 4


def _tc_body(x_ref, wpw_ref, wpb_ref, c_ref, thw_ref, thb_ref, out_ref,
             d2_ref, h_ref):
    for s in range(SLICES):
        _slice_pipe(s, x_ref, wpw_ref, wpb_ref, c_ref, thw_ref, thb_ref,
                    out_ref, d2_ref, h_ref)


def _slice_pipe(s, x_ref, wpw_ref, wpb_ref, c_ref, thw_ref, thb_ref, out_ref,
                d2_ref, h_ref):
    xb = x_ref[s]
    Z = _bdot(xb, wpw_ref[...], ((1,), (1,))) + wpb_ref[...]

    ones_col = jnp.ones((N, 1), dtype=jnp.float32)
    sq = jnp.sum(Z * Z, axis=1, keepdims=True)
    G = _bdot(Z, Z, ((1,), (1,)))
    sq_row = jnp.transpose(sq)
    d2_ref[s] = (sq - 2.0 * G) + sq_row

    INF = jnp.float32(jnp.inf)

    def _ce(u, v):
        return jnp.minimum(u, v), jnp.maximum(u, v)

    def _sorted4(t):
        q = N // 4
        a, b = _ce(t[:, :q], t[:, q:2 * q])
        c, d = _ce(t[:, 2 * q:3 * q], t[:, 3 * q:])
        a, c = _ce(a, c)
        b, d = _ce(b, d)
        b, c = _ce(b, c)
        w = q // 2
        while w >= 128:
            t1 = jnp.minimum(a[:, :w], d[:, w:])
            t2 = jnp.minimum(b[:, :w], c[:, w:])
            t3 = jnp.minimum(c[:, :w], b[:, w:])
            t4 = jnp.minimum(d[:, :w], a[:, w:])
            t1, t3 = _ce(t1, t3)
            t2, t4 = _ce(t2, t4)
            a, b = _ce(t1, t2)
            c, d = _ce(t3, t4)
            w //= 2
        return a, b, c, d

    def _extract(l1, l2, l3, l4, nstat):
        m = None
        for _ in range(nstat):
            m = jnp.min(l1, axis=1, keepdims=True)
            sh = l1 == m
            l1 = jnp.where(sh, l2, l1)
            l2 = jnp.where(sh, l3, l2)
            l3 = jnp.where(sh, l4, l3)
            l4 = jnp.where(sh, INF, l4)
        return m

    d2v = d2_ref[s]
    m = _extract(*_sorted4(d2v), 4)
    m = _extract(*_sorted4(jnp.where(d2v <= m, INF, d2v)), 4)
    m = _extract(*_sorted4(jnp.where(d2v <= m, INF, d2v)), 2)
    h_ref[s] = (d2v <= m).astype(jnp.bfloat16)

    S = _bdot(Z, c_ref[...], ((1,), (1,)))
    mx = jnp.max(S, axis=1, keepdims=True)
    e = jnp.exp(S - mx)
    Hc = e / jnp.sum(e, axis=1, keepdims=True)

    dv = jnp.float32(K) + jnp.sum(Hc, axis=1, keepdims=True)
    dvis = jax.lax.rsqrt(dv)

    Xt = _bdot(xb, thw_ref[...], ((1,), (1,))) + thb_ref[...]
    Xs = dvis * Xt

    h = h_ref[s]
    E_knn = _bdot(h, Xs, ((0,), (0,)))
    de_knn = _bdot(h, ones_col, ((0,), (0,)))
    E_knn = E_knn * jnp.where(de_knn > 0, 1.0 / jnp.where(de_knn > 0, de_knn, 1.0), 0.0)

    E_c = _bdot(Hc, Xs, ((0,), (0,)))
    de_c = _dot(Hc, ones_col, ((0,), (0,)))
    E_c = E_c * jnp.where(de_c > 0, 1.0 / jnp.where(de_c > 0, de_c, 1.0), 0.0)

    o = _bdot(h, E_knn, ((1,), (0,))) + _bdot(Hc, E_c, ((1,), (0,)))
    o = dvis * o
    out_ref[s] = jnp.where(o > 0, o, jnp.exp(jnp.minimum(o, 0.0)) - 1.0)


def kernel(x, Wp_w, Wp_b, C, Th_w, Th_b):
    Bb, Nn, Mm, Dd = x.shape
    BM = Bb * Mm
    x_flat = jnp.transpose(x, (0, 2, 1, 3)).reshape(BM, Nn, Dd)
    wpb = Wp_b.reshape(1, P)
    thb = Th_b.reshape(1, OUT)

    bcast = lambda *shape: pl.BlockSpec(shape, lambda i: (0,) * len(shape))
    y = pl.pallas_call(
        _tc_body,
        grid=(BM // SLICES,),
        in_specs=[
            pl.BlockSpec((SLICES, N, D), lambda i: (i, 0, 0)),
            bcast(P, D),
            bcast(1, P),
            bcast(NC, P),
            bcast(OUT, D),
            bcast(1, OUT),
        ],
        out_specs=pl.BlockSpec((SLICES, N, OUT), lambda i: (i, 0, 0)),
        out_shape=jax.ShapeDtypeStruct((BM, N, OUT), jnp.float32),
        scratch_shapes=[
            pltpu.VMEM((SLICES, N, N), jnp.float32),
            pltpu.VMEM((SLICES, N, N), jnp.bfloat16),
        ],
        compiler_params=pltpu.CompilerParams(
            dimension_semantics=("arbitrary",),
        ),
    )(x_flat, Wp_w, wpb, C, Th_w, thb)

    return jnp.transpose(y.reshape(Bb, Mm, Nn, OUT), (0, 2, 1, 3))


# speedup vs baseline: 1.0250x; 1.0250x over previous
import jax
import jax.numpy as jnp
from jax.experimental import pallas as pl
from jax.experimental.pallas import tpu as pltpu

N = 1024
D = 128
P = 64
NC = 10
K = 10
OUT = 128

_HIGH = jax.lax.Precision.HIGHEST


def _dot(a, b, dims):
    return jax.lax.dot_general(a, b, (dims, ((), ())),
                               preferred_element_type=jnp.float32,
                               precision=_HIGH)


def _bdot(a, b, dims):
    return jax.lax.dot_general(a.astype(jnp.bfloat16), b.astype(jnp.bfloat16),
                               (dims, ((), ())),
                               preferred_element_type=jnp.float32)


SLICES = 2


def _tc_body(x_ref, wpw_ref, wpb_ref, c_ref, thw_ref, thb_ref, out_ref,
             d2_ref, h_ref):
    for s in range(SLICES):
        _slice_pipe(s, x_ref, wpw_ref, wpb_ref, c_ref, thw_ref, thb_ref,
                    out_ref, d2_ref, h_ref)


def _slice_pipe(s, x_ref, wpw_ref, wpb_ref, c_ref, thw_ref, thb_ref, out_ref,
                d2_ref, h_ref):
    xb = x_ref[s]
    Z = _bdot(xb, wpw_ref[...], ((1,), (1,))) + wpb_ref[...]

    ones_col = jnp.ones((N, 1), dtype=jnp.float32)
    sq = jnp.sum(Z * Z, axis=1, keepdims=True)
    G = _bdot(Z, Z, ((1,), (1,)))
    sq_row = jnp.transpose(sq)
    d2_ref[s] = (sq - 2.0 * G) + sq_row

    INF = jnp.float32(jnp.inf)

    def _ce(u, v):
        return jnp.minimum(u, v), jnp.maximum(u, v)

    def _sorted4(t):
        q = N // 4
        a, b = _ce(t[:, :q], t[:, q:2 * q])
        c, d = _ce(t[:, 2 * q:3 * q], t[:, 3 * q:])
        a, c = _ce(a, c)
        b, d = _ce(b, d)
        b, c = _ce(b, c)
        w = q // 2
        while w >= 128:
            t1 = jnp.minimum(a[:, :w], d[:, w:])
            t2 = jnp.minimum(b[:, :w], c[:, w:])
            t3 = jnp.minimum(c[:, :w], b[:, w:])
            t4 = jnp.minimum(d[:, :w], a[:, w:])
            t1, t3 = _ce(t1, t3)
            t2, t4 = _ce(t2, t4)
            a, b = _ce(t1, t2)
            c, d = _ce(t3, t4)
            w //= 2
        return a, b, c, d

    def _extract(l1, l2, l3, l4, nstat):
        m = None
        for _ in range(nstat):
            m = jnp.min(l1, axis=1, keepdims=True)
            sh = l1 == m
            l1 = jnp.where(sh, l2, l1)
            l2 = jnp.where(sh, l3, l2)
            l3 = jnp.where(sh, l4, l3)
            l4 = jnp.where(sh, INF, l4)
        return m

    d2v = d2_ref[s]
    m = _extract(*_sorted4(d2v), 4)
    m = _extract(*_sorted4(jnp.where(d2v <= m, INF, d2v)), 4)
    m = _extract(*_sorted4(jnp.where(d2v <= m, INF, d2v)), 2)
    h_ref[s] = (d2v <= m).astype(jnp.bfloat16)

    S = _bdot(Z, c_ref[...], ((1,), (1,)))
    mx = jnp.max(S, axis=1, keepdims=True)
    e = jnp.exp(S - mx)
    Hc = e / jnp.sum(e, axis=1, keepdims=True)

    dv = jnp.float32(K) + jnp.sum(Hc, axis=1, keepdims=True)
    dvis = jax.lax.rsqrt(dv)

    Xt = _bdot(xb, thw_ref[...], ((1,), (1,))) + thb_ref[...]
    Xs = dvis * Xt

    h = h_ref[s]
    E_knn = _bdot(h, Xs, ((0,), (0,)))
    de_knn = _bdot(h, ones_col, ((0,), (0,)))
    E_knn = E_knn * jnp.where(de_knn > 0, 1.0 / jnp.where(de_knn > 0, de_knn, 1.0), 0.0)

    E_c = _bdot(Hc, Xs, ((0,), (0,)))
    de_c = _dot(Hc, ones_col, ((0,), (0,)))
    E_c = E_c * jnp.where(de_c > 0, 1.0 / jnp.where(de_c > 0, de_c, 1.0), 0.0)

    o = _bdot(h, E_knn, ((1,), (0,))) + _bdot(Hc, E_c, ((1,), (0,)))
    o = dvis * o
    out_ref[s] = jnp.where(o > 0, o, jnp.exp(jnp.minimum(o, 0.0)) - 1.0)


def kernel(x, Wp_w, Wp_b, C, Th_w, Th_b):
    Bb, Nn, Mm, Dd = x.shape
    BM = Bb * Mm
    x_flat = jnp.transpose(x, (0, 2, 1, 3)).reshape(BM, Nn, Dd)
    wpb = Wp_b.reshape(1, P)
    thb = Th_b.reshape(1, OUT)

    bcast = lambda *shape: pl.BlockSpec(shape, lambda i: (0,) * len(shape))
    y = pl.pallas_call(
        _tc_body,
        grid=(BM // SLICES,),
        in_specs=[
            pl.BlockSpec((SLICES, N, D), lambda i: (i, 0, 0)),
            bcast(P, D),
            bcast(1, P),
            bcast(NC, P),
            bcast(OUT, D),
            bcast(1, OUT),
        ],
        out_specs=pl.BlockSpec((SLICES, N, OUT), lambda i: (i, 0, 0)),
        out_shape=jax.ShapeDtypeStruct((BM, N, OUT), jnp.float32),
        scratch_shapes=[
            pltpu.VMEM((SLICES, N, N), jnp.float32),
            pltpu.VMEM((SLICES, N, N), jnp.bfloat16),
        ],
        compiler_params=pltpu.CompilerParams(
            dimension_semantics=("arbitrary",),
        ),
    )(x_flat, Wp_w, wpb, C, Th_w, thb)

    return jnp.transpose(y.reshape(Bb, Mm, Nn, OUT), (0, 2, 1, 3))
